# Initial kernel scaffold; baseline (speedup 1.0000x reference)
import jax
import jax.numpy as jnp
from jax.experimental import pallas as pl


def kernel(seed_points, sphere, W1, b1, W2, fa_map):
    raise NotImplementedError("write your pallas kernel here")


# speedup vs baseline: 4.0140x; 4.0140x over previous
import functools

import jax
import jax.numpy as jnp
from jax import lax
from jax.experimental import pallas as pl
from jax.experimental.pallas import tpu as pltpu
from jax.experimental.pallas import tpu_sc as plsc

B = 1024
S = 64
NSPH = 724
NSPH_PAD = 768
H = 64
VOL = 96
STEP_SIZE = 0.5
FA_TH = 0.15
BLK = 128
NW = 32
GPW = (S * B) // NW // 128


def _traj_body(seedsT_ref, w1t_ref, b1c_ref, w2t_ref, mask_ref, sph_ref,
               q_ref, dots_ref, lin_ref):
    posT0 = seedsT_ref[...]
    q_ref[0] = posT0
    lin_ref[S - 1] = jnp.zeros((1, BLK), jnp.int32)
    dots_ref[S - 1] = jnp.zeros((1, BLK), jnp.float32)
    row_iota = lax.broadcasted_iota(jnp.int32, (NSPH_PAD, BLK), 0)

    def body(t, carry):
        posT, prevT = carry
        hT = jnp.maximum(
            jnp.dot(w1t_ref[...], posT, preferred_element_type=jnp.float32)
            + b1c_ref[...], 0.0)
        sc = jnp.dot(w2t_ref[...], hT,
                     preferred_element_type=jnp.float32) + mask_ref[...]
        m = jnp.max(sc, axis=0, keepdims=True)
        idx = jnp.min(jnp.where(sc == m, row_iota, NSPH_PAD),
                      axis=0, keepdims=True)
        onehot = (row_iota == idx).astype(jnp.float32)
        ndT = lax.dot_general(sph_ref[...], onehot, (((1,), (0,)), ((), ())),
                              precision=lax.Precision.HIGHEST,
                              preferred_element_type=jnp.float32)
        dots_ref[t] = jnp.sum(ndT * prevT, axis=0, keepdims=True)
        vox = jnp.clip(jnp.round(posT).astype(jnp.int32), 0, VOL - 1)
        lin_ref[t] = (vox[0:1] * (VOL * VOL) + vox[1:2] * VOL + vox[2:3])
        posT_next = posT + STEP_SIZE * ndT
        q_ref[t + 1] = posT_next
        return posT_next, ndT

    lax.fori_loop(0, S - 1, body,
                  (posT0, jnp.zeros((3, BLK), jnp.float32)))


def _finish_body(cth_ref, q_ref, dots_ref, fa_ref, qm_ref, len_ref):
    rowi = lax.broadcasted_iota(jnp.int32, (S, B), 0)
    term = ((fa_ref[...] < FA_TH)
            | ((dots_ref[...] < cth_ref[0, 0]) & (rowi > 0))) & (rowi < S - 1)
    first = jnp.min(jnp.where(term, rowi, S - 1), axis=0, keepdims=True)
    len_ref[...] = first + 1

    def body(t, _):
        qm_ref[t] = jnp.where(first >= t, q_ref[t], 0.0)
        return 0

    lax.fori_loop(0, S, body, 0)


def _sc_gather_fn():
    mesh = plsc.VectorSubcoreMesh(core_axis_name="c", subcore_axis_name="s")

    @functools.partial(
        pl.kernel, mesh=mesh,
        out_type=jax.ShapeDtypeStruct((NW, GPW, 128), jnp.float32),
        scratch_types=[
            pltpu.VMEM((GPW, 128), jnp.int32),
            pltpu.VMEM((GPW, 128), jnp.float32),
            pltpu.SemaphoreType.DMA,
        ],
    )
    def gather_k(lin_hbm, fa_hbm, out_hbm, idx_v, vals_v, sem):
        wid = lax.axis_index("s") * 2 + lax.axis_index("c")
        pltpu.sync_copy(lin_hbm.at[wid], idx_v)
        copies = [pltpu.async_copy(fa_hbm.at[idx_v.at[j]], vals_v.at[j], sem)
                  for j in range(GPW)]
        for c in copies:
            c.wait()
        pltpu.sync_copy(vals_v, out_hbm.at[wid])

    return gather_k


_sc_gather = _sc_gather_fn()


@jax.jit
def kernel(seed_points, sphere, W1, b1, W2, fa_map):
    seedsT = seed_points[:, 0, :].T
    w1t = W1.T
    b1c = b1.reshape(H, 1)
    w2t = jnp.zeros((NSPH_PAD, H), jnp.float32).at[:NSPH].set(W2.T)
    maskc = jnp.where(jnp.arange(NSPH_PAD) < NSPH, 0.0,
                      -1e30).astype(jnp.float32).reshape(NSPH_PAD, 1)
    sph = jnp.zeros((3, NSPH_PAD), jnp.float32).at[:, :NSPH].set(sphere.T)
    cth = jnp.cos(jnp.deg2rad(jnp.float32(60.0))).reshape(1, 1)

    grid = B // BLK
    Q, dots, lin = pl.pallas_call(
        _traj_body,
        grid=(grid,),
        in_specs=[
            pl.BlockSpec((3, BLK), lambda i: (0, i)),
            pl.BlockSpec((H, 3), lambda i: (0, 0)),
            pl.BlockSpec((H, 1), lambda i: (0, 0)),
            pl.BlockSpec((NSPH_PAD, H), lambda i: (0, 0)),
            pl.BlockSpec((NSPH_PAD, 1), lambda i: (0, 0)),
            pl.BlockSpec((3, NSPH_PAD), lambda i: (0, 0)),
        ],
        out_specs=[
            pl.BlockSpec((S, 3, BLK), lambda i: (0, 0, i)),
            pl.BlockSpec((S, 1, BLK), lambda i: (0, 0, i)),
            pl.BlockSpec((S, 1, BLK), lambda i: (0, 0, i)),
        ],
        out_shape=[
            jax.ShapeDtypeStruct((S, 3, B), jnp.float32),
            jax.ShapeDtypeStruct((S, 1, B), jnp.float32),
            jax.ShapeDtypeStruct((S, 1, B), jnp.int32),
        ],
    )(seedsT, w1t, b1c, w2t, maskc, sph)

    lin32 = lin.reshape(NW, GPW, 128)
    fa = _sc_gather(lin32, fa_map.reshape(-1)).reshape(S, B)

    Qm, lens = pl.pallas_call(
        _finish_body,
        in_specs=[
            pl.BlockSpec(memory_space=pltpu.SMEM),
            pl.BlockSpec((S, 3, B), lambda: (0, 0, 0)),
            pl.BlockSpec((S, B), lambda: (0, 0)),
            pl.BlockSpec((S, B), lambda: (0, 0)),
        ],
        out_specs=[
            pl.BlockSpec((S, 3, B), lambda: (0, 0, 0)),
            pl.BlockSpec((1, B), lambda: (0, 0)),
        ],
        out_shape=[
            jax.ShapeDtypeStruct((S, 3, B), jnp.float32),
            jax.ShapeDtypeStruct((1, B), jnp.int32),
        ],
    )(cth, Q, dots.reshape(S, B), fa)

    streamlines = jnp.transpose(Qm, (2, 0, 1))
    return streamlines, lens.reshape(B)


# speedup vs baseline: 5.0964x; 1.2697x over previous
import functools

import jax
import jax.numpy as jnp
from jax import lax
from jax.experimental import pallas as pl
from jax.experimental.pallas import tpu as pltpu
from jax.experimental.pallas import tpu_sc as plsc

B = 1024
S = 64
NSPH = 724
NSPH_PAD = 768
H = 64
VOL = 96
STEP_SIZE = 0.5
FA_TH = 0.15
BLK = 128
NW = 32
GPW = (S * B) // NW // 128


def _traj_body(seedsT_ref, w1t_ref, b1c_ref, w2t_ref, mask_ref, sph_ref,
               q_ref, dots_ref, lin_ref):
    posT0 = seedsT_ref[...]
    q_ref[0] = posT0
    lin_ref[S - 1] = jnp.zeros((1, B), jnp.int32)
    dots_ref[S - 1] = jnp.zeros((1, B), jnp.float32)
    row_iota = lax.broadcasted_iota(jnp.int32, (NSPH_PAD, BLK), 0)
    nchunks = B // BLK

    def body(t, carry):
        posT, prevT = carry
        nds = []
        for c in range(nchunks):
            pc = lax.slice(posT, (0, c * BLK), (3, (c + 1) * BLK))
            hT = jnp.maximum(
                jnp.dot(w1t_ref[...], pc, preferred_element_type=jnp.float32)
                + b1c_ref[...], 0.0)
            sc = jnp.dot(w2t_ref[...], hT,
                         preferred_element_type=jnp.float32) + mask_ref[...]
            m = jnp.max(sc, axis=0, keepdims=True)
            idx = jnp.min(jnp.where(sc == m, row_iota, NSPH_PAD),
                          axis=0, keepdims=True)
            onehot = (row_iota == idx).astype(jnp.float32)
            nds.append(
                lax.dot_general(sph_ref[...], onehot, (((1,), (0,)), ((), ())),
                                precision=lax.Precision.HIGHEST,
                                preferred_element_type=jnp.float32))
        ndT = jnp.concatenate(nds, axis=1)
        dots_ref[t] = jnp.sum(ndT * prevT, axis=0, keepdims=True)
        vox = jnp.clip(jnp.round(posT).astype(jnp.int32), 0, VOL - 1)
        lin_ref[t] = (vox[0:1] * (VOL * VOL) + vox[1:2] * VOL + vox[2:3])
        posT_next = posT + STEP_SIZE * ndT
        q_ref[t + 1] = posT_next
        return posT_next, ndT

    lax.fori_loop(0, S - 1, body,
                  (posT0, jnp.zeros((3, B), jnp.float32)))


def _finish_body(cth_ref, q_ref, dots_ref, fa_ref, qm_ref, len_ref):
    rowi = lax.broadcasted_iota(jnp.int32, (S, B), 0)
    term = ((fa_ref[...] < FA_TH)
            | ((dots_ref[...] < cth_ref[0, 0]) & (rowi > 0))) & (rowi < S - 1)
    first = jnp.min(jnp.where(term, rowi, S - 1), axis=0, keepdims=True)
    len_ref[...] = first + 1

    def body(t, _):
        qm_ref[t] = jnp.where(first >= t, q_ref[t], 0.0)
        return 0

    lax.fori_loop(0, S, body, 0)


def _sc_gather_fn():
    mesh = plsc.VectorSubcoreMesh(core_axis_name="c", subcore_axis_name="s")

    @functools.partial(
        pl.kernel, mesh=mesh,
        out_type=jax.ShapeDtypeStruct((NW, GPW, 128), jnp.float32),
        scratch_types=[
            pltpu.VMEM((GPW, 128), jnp.int32),
            pltpu.VMEM((GPW, 128), jnp.float32),
            pltpu.SemaphoreType.DMA,
        ],
    )
    def gather_k(lin_hbm, fa_hbm, out_hbm, idx_v, vals_v, sem):
        wid = lax.axis_index("s") * 2 + lax.axis_index("c")
        pltpu.sync_copy(lin_hbm.at[wid], idx_v)
        copies = [pltpu.async_copy(fa_hbm.at[idx_v.at[j]], vals_v.at[j], sem)
                  for j in range(GPW)]
        for c in copies:
            c.wait()
        pltpu.sync_copy(vals_v, out_hbm.at[wid])

    return gather_k


_sc_gather = _sc_gather_fn()


@jax.jit
def kernel(seed_points, sphere, W1, b1, W2, fa_map):
    seedsT = seed_points[:, 0, :].T
    w1t = W1.T
    b1c = b1.reshape(H, 1)
    w2t = jnp.zeros((NSPH_PAD, H), jnp.float32).at[:NSPH].set(W2.T)
    maskc = jnp.where(jnp.arange(NSPH_PAD) < NSPH, 0.0,
                      -1e30).astype(jnp.float32).reshape(NSPH_PAD, 1)
    sph = jnp.zeros((3, NSPH_PAD), jnp.float32).at[:, :NSPH].set(sphere.T)
    cth = jnp.cos(jnp.deg2rad(jnp.float32(60.0))).reshape(1, 1)

    Q, dots, lin = pl.pallas_call(
        _traj_body,
        in_specs=[
            pl.BlockSpec((3, B), lambda: (0, 0)),
            pl.BlockSpec((H, 3), lambda: (0, 0)),
            pl.BlockSpec((H, 1), lambda: (0, 0)),
            pl.BlockSpec((NSPH_PAD, H), lambda: (0, 0)),
            pl.BlockSpec((NSPH_PAD, 1), lambda: (0, 0)),
            pl.BlockSpec((3, NSPH_PAD), lambda: (0, 0)),
        ],
        out_specs=[
            pl.BlockSpec((S, 3, B), lambda: (0, 0, 0)),
            pl.BlockSpec((S, 1, B), lambda: (0, 0, 0)),
            pl.BlockSpec((S, 1, B), lambda: (0, 0, 0)),
        ],
        out_shape=[
            jax.ShapeDtypeStruct((S, 3, B), jnp.float32),
            jax.ShapeDtypeStruct((S, 1, B), jnp.float32),
            jax.ShapeDtypeStruct((S, 1, B), jnp.int32),
        ],
    )(seedsT, w1t, b1c, w2t, maskc, sph)

    lin32 = lin.reshape(NW, GPW, 128)
    fa = _sc_gather(lin32, fa_map.reshape(-1)).reshape(S, B)

    Qm, lens = pl.pallas_call(
        _finish_body,
        in_specs=[
            pl.BlockSpec(memory_space=pltpu.SMEM),
            pl.BlockSpec((S, 3, B), lambda: (0, 0, 0)),
            pl.BlockSpec((S, B), lambda: (0, 0)),
            pl.BlockSpec((S, B), lambda: (0, 0)),
        ],
        out_specs=[
            pl.BlockSpec((S, 3, B), lambda: (0, 0, 0)),
            pl.BlockSpec((1, B), lambda: (0, 0)),
        ],
        out_shape=[
            jax.ShapeDtypeStruct((S, 3, B), jnp.float32),
            jax.ShapeDtypeStruct((1, B), jnp.int32),
        ],
    )(cth, Q, dots.reshape(S, B), fa)

    streamlines = jnp.transpose(Qm, (2, 0, 1))
    return streamlines, lens.reshape(B)


# speedup vs baseline: 6.7865x; 1.3316x over previous
import functools

import jax
import jax.numpy as jnp
from jax import lax
from jax.experimental import pallas as pl
from jax.experimental.pallas import tpu as pltpu
from jax.experimental.pallas import tpu_sc as plsc

B = 1024
S = 64
NSPH = 724
NSPH_PAD = 768
H = 64
VOL = 96
STEP_SIZE = 0.5
FA_TH = 0.15
BLK = 128
NW = 32
GPW = (S * B) // NW // 128


def _traj_body(seedsT_ref, w1t_ref, b1c_ref, w2t_ref, mask_ref, sph_ref,
               q_ref, dots_ref, lin_ref):
    posT0 = seedsT_ref[...]
    q_ref[0] = posT0
    lin_ref[S - 1] = jnp.zeros((1, B), jnp.int32)
    dots_ref[S - 1] = jnp.zeros((1, B), jnp.float32)
    row_iota = lax.broadcasted_iota(jnp.int32, (NSPH_PAD, BLK), 0)
    nchunks = B // BLK

    def body(t, carry):
        posT, prevT = carry
        nds = []
        for c in range(nchunks):
            pc = lax.slice(posT, (0, c * BLK), (3, (c + 1) * BLK))
            hT = jnp.maximum(
                jnp.dot(w1t_ref[...], pc, preferred_element_type=jnp.float32)
                + b1c_ref[...], 0.0)
            sc = jnp.dot(w2t_ref[...], hT,
                         preferred_element_type=jnp.float32) + mask_ref[...]
            idx = jnp.argmax(sc, axis=0)[None, :]
            onehot = (row_iota == idx).astype(jnp.float32)
            nds.append(
                lax.dot_general(sph_ref[...], onehot, (((1,), (0,)), ((), ())),
                                preferred_element_type=jnp.float32))
        ndT = jnp.concatenate(nds, axis=1)
        dots_ref[t] = jnp.sum(ndT * prevT, axis=0, keepdims=True)
        vox = jnp.clip(jnp.round(posT).astype(jnp.int32), 0, VOL - 1)
        lin_ref[t] = (vox[0:1] * (VOL * VOL) + vox[1:2] * VOL + vox[2:3])
        posT_next = posT + STEP_SIZE * ndT
        q_ref[t + 1] = posT_next
        return posT_next, ndT

    lax.fori_loop(0, S - 1, body,
                  (posT0, jnp.zeros((3, B), jnp.float32)))


def _finish_body(cth_ref, q_ref, dots_ref, fa_ref, qm_ref, len_ref):
    rowi = lax.broadcasted_iota(jnp.int32, (S, B), 0)
    term = ((fa_ref[...] < FA_TH)
            | ((dots_ref[...] < cth_ref[0, 0]) & (rowi > 0))) & (rowi < S - 1)
    first = jnp.min(jnp.where(term, rowi, S - 1), axis=0, keepdims=True)
    len_ref[...] = first + 1

    def body(t, _):
        qm_ref[t] = jnp.where(first >= t, q_ref[t], 0.0)
        return 0

    lax.fori_loop(0, S, body, 0)


def _sc_gather_fn():
    mesh = plsc.VectorSubcoreMesh(core_axis_name="c", subcore_axis_name="s")

    @functools.partial(
        pl.kernel, mesh=mesh,
        out_type=jax.ShapeDtypeStruct((NW, GPW, 128), jnp.float32),
        scratch_types=[
            pltpu.VMEM((GPW, 128), jnp.int32),
            pltpu.VMEM((GPW, 128), jnp.float32),
            pltpu.SemaphoreType.DMA,
        ],
    )
    def gather_k(lin_hbm, fa_hbm, out_hbm, idx_v, vals_v, sem):
        wid = lax.axis_index("s") * 2 + lax.axis_index("c")
        pltpu.sync_copy(lin_hbm.at[wid], idx_v)
        copies = [pltpu.async_copy(fa_hbm.at[idx_v.at[j]], vals_v.at[j], sem)
                  for j in range(GPW)]
        for c in copies:
            c.wait()
        pltpu.sync_copy(vals_v, out_hbm.at[wid])

    return gather_k


_sc_gather = _sc_gather_fn()


@jax.jit
def kernel(seed_points, sphere, W1, b1, W2, fa_map):
    seedsT = seed_points[:, 0, :].T
    w1t = W1.T
    b1c = b1.reshape(H, 1)
    w2t = jnp.zeros((NSPH_PAD, H), jnp.float32).at[:NSPH].set(W2.T)
    maskc = jnp.where(jnp.arange(NSPH_PAD) < NSPH, 0.0,
                      -1e30).astype(jnp.float32).reshape(NSPH_PAD, 1)
    sph = jnp.zeros((3, NSPH_PAD), jnp.float32).at[:, :NSPH].set(sphere.T)
    cth = jnp.cos(jnp.deg2rad(jnp.float32(60.0))).reshape(1, 1)

    Q, dots, lin = pl.pallas_call(
        _traj_body,
        in_specs=[
            pl.BlockSpec((3, B), lambda: (0, 0)),
            pl.BlockSpec((H, 3), lambda: (0, 0)),
            pl.BlockSpec((H, 1), lambda: (0, 0)),
            pl.BlockSpec((NSPH_PAD, H), lambda: (0, 0)),
            pl.BlockSpec((NSPH_PAD, 1), lambda: (0, 0)),
            pl.BlockSpec((3, NSPH_PAD), lambda: (0, 0)),
        ],
        out_specs=[
            pl.BlockSpec((S, 3, B), lambda: (0, 0, 0)),
            pl.BlockSpec((S, 1, B), lambda: (0, 0, 0)),
            pl.BlockSpec((S, 1, B), lambda: (0, 0, 0)),
        ],
        out_shape=[
            jax.ShapeDtypeStruct((S, 3, B), jnp.float32),
            jax.ShapeDtypeStruct((S, 1, B), jnp.float32),
            jax.ShapeDtypeStruct((S, 1, B), jnp.int32),
        ],
    )(seedsT, w1t, b1c, w2t, maskc, sph)

    lin32 = lin.reshape(NW, GPW, 128)
    fa = _sc_gather(lin32, fa_map.reshape(-1)).reshape(S, B)

    Qm, lens = pl.pallas_call(
        _finish_body,
        in_specs=[
            pl.BlockSpec(memory_space=pltpu.SMEM),
            pl.BlockSpec((S, 3, B), lambda: (0, 0, 0)),
            pl.BlockSpec((S, B), lambda: (0, 0)),
            pl.BlockSpec((S, B), lambda: (0, 0)),
        ],
        out_specs=[
            pl.BlockSpec((S, 3, B), lambda: (0, 0, 0)),
            pl.BlockSpec((1, B), lambda: (0, 0)),
        ],
        out_shape=[
            jax.ShapeDtypeStruct((S, 3, B), jnp.float32),
            jax.ShapeDtypeStruct((1, B), jnp.int32),
        ],
    )(cth, Q, dots.reshape(S, B), fa)

    streamlines = jnp.transpose(Qm, (2, 0, 1))
    return streamlines, lens.reshape(B)
